# baseline (speedup 1.0000x reference)
import functools

import jax
import jax.numpy as jnp
from jax import lax
from jax.experimental import pallas as pl
from jax.experimental.pallas import tpu as pltpu
from jax.experimental.pallas import tpu_sc as plsc

N_POS_K = 4096
N_NEG_K = 12288
THRESH_K = 1.0

_NUM_WORKERS = 32
_BPW = N_POS_K // _NUM_WORKERS
_L = 16

_TILE_P = 1024
_TILE_N = 12288


def _sc_corr_body(idx_hbm, tab_hbm, af_hbm, fns_hbm, out_hbm,
                  idx_v, lam_v, af_v, negf_v, acc_v, sem):
    wid = lax.axis_index("s") * 2 + lax.axis_index("c")
    base = wid * _BPW

    pltpu.sync_copy(idx_hbm.at[pl.ds(base, _BPW)], idx_v)
    pltpu.async_copy(tab_hbm.at[idx_v], lam_v, sem).wait()

    acc_v[...] = jnp.zeros((_L,), jnp.float32)

    def _any_chunk(k, m):
        return jnp.maximum(m, jnp.where(lam_v[pl.ds(k * _L, _L)] > 0.0, 1.0, 0.0))

    flag16 = lax.fori_loop(0, _BPW // _L, _any_chunk,
                           jnp.zeros((_L,), jnp.float32))
    any_pos = sum(flag16[k] for k in range(_L)) > 0.5

    @pl.when(any_pos)
    def _correct():
        pltpu.sync_copy(af_hbm.at[pl.ds(base, _BPW)], af_v)
        pltpu.sync_copy(fns_hbm, negf_v)

        def _rowgroup(g, carry):
            sl = pl.ds(g * _L, _L)
            lamg = lam_v[sl]
            ag = af_v[sl]
            for k in range(_L):
                lam_r = lamg[k]
                lam16 = jnp.full((_L,), lam_r)
                a16 = jnp.full((_L,), ag[k])

                @pl.when(lam_r > 0.0)
                def _do_row(a16=a16, lam16=lam16):
                    def _chunk(c, acc2):
                        n = negf_v[pl.ds(c * _L, _L)]
                        t = a16 + n
                        m = jnp.maximum(t, 0.0)
                        h = m * m
                        keep = (t > 0.0) & (h <= lam16)
                        return acc2 + jnp.where(keep, h, 0.0)

                    row16 = lax.fori_loop(0, N_NEG_K // _L, _chunk,
                                          jnp.zeros((_L,), jnp.float32))
                    acc_v[...] = acc_v[...] + row16

            return carry

        lax.fori_loop(0, _BPW // _L, _rowgroup, 0)

    pltpu.sync_copy(acc_v, out_hbm.at[wid])


def _sc_correction(idx, table, af, fnsb):
    mesh = plsc.VectorSubcoreMesh(core_axis_name="c", subcore_axis_name="s")
    k = pl.kernel(
        _sc_corr_body,
        out_type=jax.ShapeDtypeStruct((_NUM_WORKERS, _L), jnp.float32),
        mesh=mesh,
        scratch_types=[
            pltpu.VMEM((_BPW,), jnp.int32),
            pltpu.VMEM((_BPW,), jnp.float32),
            pltpu.VMEM((_BPW,), jnp.float32),
            pltpu.VMEM((N_NEG_K,), jnp.float32),
            pltpu.VMEM((_L,), jnp.float32),
            pltpu.SemaphoreType.DMA,
        ],
    )
    return k(idx, table, af, fnsb)


def _tc_loss_body(yp_ref, yt_ref, fns_ref, out_ref, acc_ref):
    i = pl.program_id(0)
    j = pl.program_id(1)

    @pl.when((i == 0) & (j == 0))
    def _init():
        acc_ref[...] = jnp.zeros_like(acc_ref)

    fps = jnp.where(yt_ref[...] == 1, yp_ref[...], 0.0)
    a = (THRESH_K - fps).astype(jnp.bfloat16)
    t = a + fns_ref[...]
    m = jnp.maximum(t, jnp.bfloat16(0.0))
    masked = m * m
    ones = jnp.ones((1, _TILE_P), dtype=jnp.bfloat16)
    red = jax.lax.dot_general(ones, masked, (((1,), (0,)), ((), ())),
                              preferred_element_type=jnp.float32)
    acc_ref[...] += red

    @pl.when((i == pl.num_programs(0) - 1) & (j == pl.num_programs(1) - 1))
    def _fin():
        out_ref[...] = jnp.full((1, 1), jnp.sum(acc_ref[...]) * (1.0 / N_POS_K),
                                dtype=jnp.float32)


_tc_loss = pl.pallas_call(
    _tc_loss_body,
    grid=(N_POS_K // _TILE_P, N_NEG_K // _TILE_N),
    in_specs=[
        pl.BlockSpec((_TILE_P, 1), lambda i, j: (i, 0)),
        pl.BlockSpec((_TILE_P, 1), lambda i, j: (i, 0)),
        pl.BlockSpec((1, _TILE_N), lambda i, j: (0, j)),
    ],
    out_specs=pl.BlockSpec((1, 1), lambda i, j: (0, 0)),
    out_shape=jax.ShapeDtypeStruct((1, 1), jnp.float32),
    scratch_shapes=[pltpu.VMEM((1, _TILE_N), jnp.float32)],
)


def kernel(y_pred, y_true, index_p, lambda_pos):
    yp = y_pred[:, 0]
    yt = y_true[:, 0]

    fns_f32 = yp[N_POS_K:] * (yt[N_POS_K:] == 0)
    fns_bf = fns_f32.astype(jnp.bfloat16)
    af = THRESH_K - yp[:N_POS_K] * (yt[:N_POS_K] == 1)

    corr = _sc_correction(index_p.reshape(-1), lambda_pos.reshape(-1),
                          af, fns_f32)
    u = _tc_loss(y_pred, y_true, fns_bf.reshape(1, N_NEG_K))
    return u[0, 0] - jnp.sum(corr) * (1.0 / N_POS_K)


# speedup vs baseline: 1.2195x; 1.2195x over previous
import functools

import jax
import jax.numpy as jnp
from jax import lax
from jax.experimental import pallas as pl
from jax.experimental.pallas import tpu as pltpu
from jax.experimental.pallas import tpu_sc as plsc

N_POS_K = 4096
N_NEG_K = 12288
THRESH_K = 1.0

_NUM_WORKERS = 32
_BPW = N_POS_K // _NUM_WORKERS
_L = 16

_TILE_P = 1024
_TILE_N = 12288


def _sc_corr_body(idx_hbm, tab_hbm, af_hbm, fns_hbm, out_hbm,
                  idx_v, lam_v, af_v, negf_v, acc_v, sem):
    wid = lax.axis_index("s") * 2 + lax.axis_index("c")
    base = wid * _BPW

    pltpu.sync_copy(idx_hbm.at[pl.ds(base, _BPW)], idx_v)
    pltpu.async_copy(tab_hbm.at[idx_v], lam_v, sem).wait()

    acc_v[...] = jnp.zeros((_L,), jnp.float32)

    def _any_chunk(k, m):
        return jnp.maximum(m, jnp.where(lam_v[pl.ds(k * _L, _L)] > 0.0, 1.0, 0.0))

    flag16 = lax.fori_loop(0, _BPW // _L, _any_chunk,
                           jnp.zeros((_L,), jnp.float32))
    any_pos = sum(flag16[k] for k in range(_L)) > 0.5

    @pl.when(any_pos)
    def _correct():
        pltpu.sync_copy(af_hbm.at[pl.ds(base, _BPW)], af_v)
        pltpu.sync_copy(fns_hbm, negf_v)

        def _rowgroup(g, carry):
            sl = pl.ds(g * _L, _L)
            lamg = lam_v[sl]
            ag = af_v[sl]
            for k in range(_L):
                lam_r = lamg[k]
                lam16 = jnp.full((_L,), lam_r)
                a16 = jnp.full((_L,), ag[k])

                @pl.when(lam_r > 0.0)
                def _do_row(a16=a16, lam16=lam16):
                    def _chunk(c, acc2):
                        n = negf_v[pl.ds(c * _L, _L)]
                        t = a16 + n
                        m = jnp.maximum(t, 0.0)
                        h = m * m
                        keep = (t > 0.0) & (h <= lam16)
                        return acc2 + jnp.where(keep, h, 0.0)

                    row16 = lax.fori_loop(0, N_NEG_K // _L, _chunk,
                                          jnp.zeros((_L,), jnp.float32))
                    acc_v[...] = acc_v[...] + row16

            return carry

        lax.fori_loop(0, _BPW // _L, _rowgroup, 0)

    pltpu.sync_copy(acc_v, out_hbm.at[wid])


def _sc_correction(idx, table, af, fnsb):
    mesh = plsc.VectorSubcoreMesh(core_axis_name="c", subcore_axis_name="s")
    k = pl.kernel(
        _sc_corr_body,
        out_type=jax.ShapeDtypeStruct((_NUM_WORKERS, _L), jnp.float32),
        mesh=mesh,
        scratch_types=[
            pltpu.VMEM((_BPW,), jnp.int32),
            pltpu.VMEM((_BPW,), jnp.float32),
            pltpu.VMEM((_BPW,), jnp.float32),
            pltpu.VMEM((N_NEG_K,), jnp.float32),
            pltpu.VMEM((_L,), jnp.float32),
            pltpu.SemaphoreType.DMA,
        ],
    )
    return k(idx, table, af, fnsb)


def _tc_loss_body(a_ref, fns_ref, out_ref, acc_ref):
    i = pl.program_id(0)
    j = pl.program_id(1)

    @pl.when((i == 0) & (j == 0))
    def _init():
        acc_ref[...] = jnp.zeros_like(acc_ref)

    t = a_ref[...] + fns_ref[...]
    m = jnp.maximum(t, jnp.bfloat16(0.0))
    masked = m * m
    ones = jnp.ones((1, _TILE_P), dtype=jnp.bfloat16)
    red = jax.lax.dot_general(ones, masked, (((1,), (0,)), ((), ())),
                              preferred_element_type=jnp.float32)
    acc_ref[...] += red

    @pl.when((i == pl.num_programs(0) - 1) & (j == pl.num_programs(1) - 1))
    def _fin():
        out_ref[...] = jnp.full((1, 1), jnp.sum(acc_ref[...]) * (1.0 / N_POS_K),
                                dtype=jnp.float32)


_tc_loss = pl.pallas_call(
    _tc_loss_body,
    grid=(N_POS_K // _TILE_P, N_NEG_K // _TILE_N),
    in_specs=[
        pl.BlockSpec((_TILE_P, 1), lambda i, j: (i, 0)),
        pl.BlockSpec((1, _TILE_N), lambda i, j: (0, j)),
    ],
    out_specs=pl.BlockSpec((1, 1), lambda i, j: (0, 0)),
    out_shape=jax.ShapeDtypeStruct((1, 1), jnp.float32),
    scratch_shapes=[pltpu.VMEM((1, _TILE_N), jnp.float32)],
)


def kernel(y_pred, y_true, index_p, lambda_pos):
    yp = y_pred[:, 0]
    yt = y_true[:, 0]

    fns_f32 = yp[N_POS_K:] * (yt[N_POS_K:] == 0)
    fns_bf = fns_f32.astype(jnp.bfloat16)
    af = THRESH_K - yp[:N_POS_K] * (yt[:N_POS_K] == 1)
    a_bf = af.astype(jnp.bfloat16).reshape(N_POS_K, 1)

    corr = _sc_correction(index_p.reshape(-1), lambda_pos.reshape(-1),
                          af, fns_f32)
    u = _tc_loss(a_bf, fns_bf.reshape(1, N_NEG_K))
    return u[0, 0] - jnp.sum(corr) * (1.0 / N_POS_K)


# speedup vs baseline: 1.2214x; 1.0015x over previous
import functools

import jax
import jax.numpy as jnp
from jax import lax
from jax.experimental import pallas as pl
from jax.experimental.pallas import tpu as pltpu
from jax.experimental.pallas import tpu_sc as plsc

N_POS_K = 4096
N_NEG_K = 12288
THRESH_K = 1.0

_NUM_WORKERS = 32
_BPW = N_POS_K // _NUM_WORKERS
_L = 16

_TILE_P = 1024
_TILE_N = 12288


def _sc_corr_body(idx_hbm, tab_hbm, af_hbm, fns_hbm, out_hbm,
                  idx_v, lam_v, af_v, negf_v, acc_v, sem):
    wid = lax.axis_index("s") * 2 + lax.axis_index("c")
    base = wid * _BPW

    pltpu.sync_copy(idx_hbm.at[pl.ds(base, _BPW)], idx_v)
    pltpu.async_copy(tab_hbm.at[idx_v], lam_v, sem).wait()

    acc_v[...] = jnp.zeros((_L,), jnp.float32)

    def _any_chunk(k, m):
        return jnp.maximum(m, jnp.where(lam_v[pl.ds(k * _L, _L)] > 0.0, 1.0, 0.0))

    flag16 = lax.fori_loop(0, _BPW // _L, _any_chunk,
                           jnp.zeros((_L,), jnp.float32))
    any_pos = sum(flag16[k] for k in range(_L)) > 0.5

    @pl.when(any_pos)
    def _correct():
        pltpu.sync_copy(af_hbm.at[pl.ds(base, _BPW)], af_v)
        pltpu.sync_copy(fns_hbm, negf_v)

        def _rowgroup(g, carry):
            sl = pl.ds(g * _L, _L)
            lamg = lam_v[sl]
            ag = af_v[sl]
            for k in range(_L):
                lam_r = lamg[k]
                lam16 = jnp.full((_L,), lam_r)
                a16 = jnp.full((_L,), ag[k])

                @pl.when(lam_r > 0.0)
                def _do_row(a16=a16, lam16=lam16):
                    def _chunk(c, acc2):
                        n = negf_v[pl.ds(c * _L, _L)]
                        t = a16 + n
                        m = jnp.maximum(t, 0.0)
                        h = m * m
                        keep = (t > 0.0) & (h <= lam16)
                        return acc2 + jnp.where(keep, h, 0.0)

                    row16 = lax.fori_loop(0, N_NEG_K // _L, _chunk,
                                          jnp.zeros((_L,), jnp.float32))
                    acc_v[...] = acc_v[...] + row16

            return carry

        lax.fori_loop(0, _BPW // _L, _rowgroup, 0)

    pltpu.sync_copy(acc_v, out_hbm.at[wid])


def _sc_correction(idx, table, af, fnsb):
    mesh = plsc.VectorSubcoreMesh(core_axis_name="c", subcore_axis_name="s")
    k = pl.kernel(
        _sc_corr_body,
        out_type=jax.ShapeDtypeStruct((_NUM_WORKERS, _L), jnp.float32),
        mesh=mesh,
        scratch_types=[
            pltpu.VMEM((_BPW,), jnp.int32),
            pltpu.VMEM((_BPW,), jnp.float32),
            pltpu.VMEM((_BPW,), jnp.float32),
            pltpu.VMEM((N_NEG_K,), jnp.float32),
            pltpu.VMEM((_L,), jnp.float32),
            pltpu.SemaphoreType.DMA,
        ],
    )
    return k(idx, table, af, fnsb)


def _tc_loss_body(a_ref, fns_ref, out_ref, acc_ref):
    i = pl.program_id(0)
    j = pl.program_id(1)

    @pl.when((i == 0) & (j == 0))
    def _init():
        acc_ref[...] = jnp.zeros_like(acc_ref)

    t = a_ref[...] + fns_ref[...]
    m = jnp.maximum(t, jnp.bfloat16(0.0))
    masked = m * m
    ones = jnp.ones((1, _TILE_P), dtype=jnp.bfloat16)
    red = jax.lax.dot_general(ones, masked, (((1,), (0,)), ((), ())),
                              preferred_element_type=jnp.float32)
    acc_ref[...] += red

    @pl.when((i == pl.num_programs(0) - 1) & (j == pl.num_programs(1) - 1))
    def _fin():
        out_ref[...] = jnp.full((1, 1), jnp.sum(acc_ref[...]) * (1.0 / N_POS_K),
                                dtype=jnp.float32)


_tc_loss = pl.pallas_call(
    _tc_loss_body,
    grid=(N_POS_K // _TILE_P, N_NEG_K // _TILE_N),
    in_specs=[
        pl.BlockSpec((_TILE_P, 1), lambda i, j: (i, 0)),
        pl.BlockSpec((1, _TILE_N), lambda i, j: (0, j)),
    ],
    out_specs=pl.BlockSpec((1, 1), lambda i, j: (0, 0)),
    out_shape=jax.ShapeDtypeStruct((1, 1), jnp.float32),
    scratch_shapes=[pltpu.VMEM((1, _TILE_N), jnp.float32)],
)


def kernel(y_pred, y_true, index_p, lambda_pos):
    yp = y_pred[:, 0]
    yt = y_true[:, 0]

    fns_f32 = yp[N_POS_K:] * (yt[N_POS_K:] == 0)
    fns_bf = fns_f32.astype(jnp.bfloat16).reshape(1, N_NEG_K)
    af = THRESH_K - yp[:N_POS_K] * (yt[:N_POS_K] == 1)
    a_bf = af.astype(jnp.bfloat16).reshape(N_POS_K, 1)

    corr = _sc_correction(index_p.reshape(-1), lambda_pos.reshape(-1),
                          af, fns_f32)
    u = _tc_loss(a_bf, fns_bf)
    return u[0, 0] - jnp.sum(corr) * (1.0 / N_POS_K)
